# baseline (speedup 1.0000x reference)
import functools

import jax
import jax.numpy as jnp
from jax import lax
from jax.experimental import pallas as pl
from jax.experimental.pallas import tpu as pltpu
from jax.experimental.pallas import tpu_sc as plsc

N = 16384
A = 26
H = 64
NOBJ = 1024
K = 20
NACT = 8

AH = A * H
BN = 4096
GN = N // BN
NFULL = N + NACT + 1
NW = 32
OBJ_PER_W = NOBJ // NW
IDX_PER_W = OBJ_PER_W * K
CHUNK = 128
NCHUNK = IDX_PER_W // CHUNK

_CONTRACT0 = (((0,), (0,)), ((), ()))


def _entity_tile(valsT_ref, rtq_ref, wf_ref, bf_ref, wa_ref, ba_ref,
                 went_ref, atnT_ref, outT_ref, el2_ref):
    i = pl.program_id(0)

    vt = valsT_ref[...]
    vtb = vt.astype(jnp.bfloat16)
    v_rep = jnp.broadcast_to(vtb[:, None, :], (A, H, BN)).reshape(AH, BN)
    th = jnp.tanh(v_rep * wf_ref[...] + bf_ref[...])
    scoresT = lax.dot_general(rtq_ref[...], th, _CONTRACT0,
                              preferred_element_type=jnp.float32)
    m = jnp.max(scoresT, axis=0, keepdims=True)
    e = jnp.exp(scoresT - m)
    wT = e / jnp.sum(e, axis=0, keepdims=True)
    elT = (lax.dot_general(wa_ref[...], wT * vt, _CONTRACT0,
                           preferred_element_type=jnp.float32)
           + lax.dot_general(ba_ref[...], wT, _CONTRACT0,
                             preferred_element_type=jnp.float32))
    outT_ref[...] = elT
    el2 = lax.dot_general(elT, went_ref[...], _CONTRACT0,
                          preferred_element_type=jnp.float32)
    el2_ref[:, 0:H] = el2

    @pl.when(i == GN)
    def _():
        outT_ref[:, 0:16] = atnT_ref[...]


def _entity_lookup(attr_vals, W_attr, b_attr, q_attn, W_ent, action_table):
    valsT = attr_vals.T
    wf = W_attr.reshape(AH, 1).astype(jnp.bfloat16)
    bf = b_attr.reshape(AH, 1).astype(jnp.bfloat16)
    rtq = (jnp.eye(A, dtype=jnp.float32)[:, None, :]
           * q_attn.reshape(1, H, 1)).reshape(AH, A).astype(jnp.bfloat16)
    atnT = jnp.concatenate(
        [action_table.T, jnp.zeros((H, 16 - NACT), jnp.float32)], axis=1)
    clamp = lambda i: (0, jnp.minimum(i, GN - 1))
    full = lambda *shape: pl.BlockSpec(shape, lambda i: (0,) * len(shape))
    return pl.pallas_call(
        _entity_tile,
        grid=(GN + 1,),
        in_specs=[
            pl.BlockSpec((A, BN), clamp),
            full(AH, A), full(AH, 1), full(AH, 1),
            full(A, H), full(A, H), full(H, H), full(H, 16),
        ],
        out_specs=[
            pl.BlockSpec((H, BN), lambda i: (0, i)),
            pl.BlockSpec((BN, 2 * H), lambda i: (jnp.minimum(i, GN - 1), 0)),
        ],
        out_shape=[
            jax.ShapeDtypeStruct((H, NFULL), jnp.float32),
            jax.ShapeDtypeStruct((N, 2 * H), jnp.float32),
        ],
    )(valsT, rtq, wf, bf, W_attr, b_attr, W_ent, atnT)


_OBJ_HI = [(CHUNK * (j + 1) - K) // K + 1 for j in range(NCHUNK)]
_OBJ_LO = [0] + _OBJ_HI[:-1]


def _gather_body(el2_hbm, idx_hbm, out_hbm, idx_v, rows_v, pooled_v,
                 s0, s1, s2, s3, s4):
    wid = lax.axis_index("s") * 2 + lax.axis_index("c")
    sems = (s0, s1, s2, s3, s4)
    pltpu.sync_copy(idx_hbm.at[wid], idx_v)
    copies = [
        pltpu.async_copy(el2_hbm.at[idx_v.at[j]],
                         rows_v.at[pl.ds(j * CHUNK, CHUNK)], sems[j])
        for j in range(NCHUNK)
    ]

    inv_k = jnp.float32(1.0 / K)

    def body(o, carry):
        for c in range(H // 16):
            acc = rows_v[o * K, pl.ds(c * 16, 16)]
            for k in range(1, K):
                acc = acc + rows_v[o * K + k, pl.ds(c * 16, 16)]
            y = acc * inv_k
            t = 1.0 - 2.0 / (jnp.exp(2.0 * y) + 1.0)
            pooled_v[o, pl.ds(c * 16, 16)] = t
        return carry

    for j in range(NCHUNK):
        copies[j].wait()
        lax.fori_loop(_OBJ_LO[j], _OBJ_HI[j], body, 0, unroll=False)
    pltpu.sync_copy(pooled_v, out_hbm.at[pl.ds(wid * OBJ_PER_W, OBJ_PER_W)])


def _gather_pool(el2, idx3d):
    mesh = plsc.VectorSubcoreMesh(core_axis_name="c", subcore_axis_name="s")
    fn = functools.partial(
        pl.kernel,
        mesh=mesh,
        out_type=jax.ShapeDtypeStruct((NOBJ, H), jnp.float32),
        scratch_types=[
            pltpu.VMEM((NCHUNK, CHUNK), jnp.int32),
            pltpu.VMEM((IDX_PER_W, 2 * H), jnp.float32),
            pltpu.VMEM((OBJ_PER_W, H), jnp.float32),
        ] + [pltpu.SemaphoreType.DMA] * NCHUNK,
    )(_gather_body)
    return fn(el2, idx3d)


def kernel(attr_vals, obs_idx, W_attr, b_attr, q_attn, W_ent, action_table):
    el_fullT, el2 = _entity_lookup(attr_vals, W_attr, b_attr, q_attn,
                                   W_ent, action_table)
    idx3d = obs_idx.astype(jnp.int32).reshape(NW, NCHUNK, CHUNK)
    obs = _gather_pool(el2, idx3d)
    return obs, el_fullT.T


# speedup vs baseline: 1.0196x; 1.0196x over previous
import functools

import jax
import jax.numpy as jnp
from jax import lax
from jax.experimental import pallas as pl
from jax.experimental.pallas import tpu as pltpu
from jax.experimental.pallas import tpu_sc as plsc

N = 16384
A = 26
H = 64
NOBJ = 1024
K = 20
NACT = 8

AH = A * H
BN = 4096
GN = N // BN
NFULL = N + NACT + 1
NW = 32
OBJ_PER_W = NOBJ // NW
IDX_PER_W = OBJ_PER_W * K
CHUNK = 128
NCHUNK = IDX_PER_W // CHUNK

_CONTRACT0 = (((0,), (0,)), ((), ()))


def _entity_tile(valsT_ref, rtq_ref, wf_ref, bf_ref, wa_ref, ba_ref,
                 went_ref, atnT_ref, outT_ref, el2_ref):
    i = pl.program_id(0)

    vt = valsT_ref[...]
    vtb = vt.astype(jnp.bfloat16)
    v_rep = jnp.broadcast_to(vtb[:, None, :], (A, H, BN)).reshape(AH, BN)
    th = jnp.tanh(v_rep * wf_ref[...] + bf_ref[...])
    scoresT = lax.dot_general(rtq_ref[...], th, _CONTRACT0,
                              preferred_element_type=jnp.float32)
    m = jnp.max(scoresT, axis=0, keepdims=True)
    e = jnp.exp(scoresT - m)
    wT = e / jnp.sum(e, axis=0, keepdims=True)
    elT = (lax.dot_general(wa_ref[...], wT * vt, _CONTRACT0,
                           preferred_element_type=jnp.float32)
           + lax.dot_general(ba_ref[...], wT, _CONTRACT0,
                             preferred_element_type=jnp.float32))
    outT_ref[...] = elT
    el2 = lax.dot_general(elT, went_ref[...], _CONTRACT0,
                          preferred_element_type=jnp.float32)
    el2_ref[:, 0:H] = el2

    @pl.when(i == GN)
    def _():
        outT_ref[:, 0:16] = atnT_ref[...]


def _entity_lookup(attr_vals, W_attr, b_attr, q_attn, W_ent, action_table):
    valsT = attr_vals.T
    wf = W_attr.reshape(AH, 1).astype(jnp.bfloat16)
    bf = b_attr.reshape(AH, 1).astype(jnp.bfloat16)
    rtq = (jnp.eye(A, dtype=jnp.float32)[:, None, :]
           * q_attn.reshape(1, H, 1)).reshape(AH, A).astype(jnp.bfloat16)
    atnT = jnp.concatenate(
        [action_table.T, jnp.zeros((H, 16 - NACT), jnp.float32)], axis=1)
    clamp = lambda i: (0, jnp.minimum(i, GN - 1))
    full = lambda *shape: pl.BlockSpec(shape, lambda i: (0,) * len(shape))
    return pl.pallas_call(
        _entity_tile,
        grid=(GN + 1,),
        in_specs=[
            pl.BlockSpec((A, BN), clamp),
            full(AH, A), full(AH, 1), full(AH, 1),
            full(A, H), full(A, H), full(H, H), full(H, 16),
        ],
        out_specs=[
            pl.BlockSpec((H, BN), lambda i: (0, i)),
            pl.BlockSpec((BN, 2 * H), lambda i: (jnp.minimum(i, GN - 1), 0)),
        ],
        out_shape=[
            jax.ShapeDtypeStruct((H, NFULL), jnp.float32),
            jax.ShapeDtypeStruct((N, 2 * H), jnp.float32),
        ],
    )(valsT, rtq, wf, bf, W_attr, b_attr, W_ent, atnT)


def _gather_body(el2_hbm, idx_hbm, out_hbm, idx_v, rows_v, pooled_v, sem):
    wid = lax.axis_index("s") * 2 + lax.axis_index("c")
    pltpu.sync_copy(idx_hbm.at[wid], idx_v)
    copies = [
        pltpu.async_copy(el2_hbm.at[idx_v.at[j]],
                         rows_v.at[pl.ds(j * CHUNK, CHUNK)], sem)
        for j in range(NCHUNK)
    ]
    for c in copies:
        c.wait()

    inv_k = jnp.float32(1.0 / K)

    def body(o, carry):
        for c in range(H // 16):
            acc = rows_v[o * K, pl.ds(c * 16, 16)]
            for k in range(1, K):
                acc = acc + rows_v[o * K + k, pl.ds(c * 16, 16)]
            y = acc * inv_k
            t = 1.0 - 2.0 / (jnp.exp(2.0 * y) + 1.0)
            pooled_v[o, pl.ds(c * 16, 16)] = t
        return carry

    lax.fori_loop(0, OBJ_PER_W, body, 0, unroll=False)
    pltpu.sync_copy(pooled_v, out_hbm.at[pl.ds(wid * OBJ_PER_W, OBJ_PER_W)])


def _gather_pool(el2, idx3d):
    mesh = plsc.VectorSubcoreMesh(core_axis_name="c", subcore_axis_name="s")
    fn = functools.partial(
        pl.kernel,
        mesh=mesh,
        out_type=jax.ShapeDtypeStruct((NOBJ, H), jnp.float32),
        scratch_types=[
            pltpu.VMEM((NCHUNK, CHUNK), jnp.int32),
            pltpu.VMEM((IDX_PER_W, 2 * H), jnp.float32),
            pltpu.VMEM((OBJ_PER_W, H), jnp.float32),
            pltpu.SemaphoreType.DMA,
        ],
    )(_gather_body)
    return fn(el2, idx3d)


def kernel(attr_vals, obs_idx, W_attr, b_attr, q_attn, W_ent, action_table):
    el_fullT, el2 = _entity_lookup(attr_vals, W_attr, b_attr, q_attn,
                                   W_ent, action_table)
    idx3d = obs_idx.astype(jnp.int32).reshape(NW, NCHUNK, CHUNK)
    obs = _gather_pool(el2, idx3d)
    return obs, el_fullT.T


# speedup vs baseline: 1.0239x; 1.0041x over previous
import functools

import jax
import jax.numpy as jnp
from jax import lax
from jax.experimental import pallas as pl
from jax.experimental.pallas import tpu as pltpu
from jax.experimental.pallas import tpu_sc as plsc

N = 16384
A = 26
H = 64
NOBJ = 1024
K = 20
NACT = 8

AH = A * H
BN = 4096
GN = N // BN
NFULL = N + NACT + 1
NW = 32
OBJ_PER_W = NOBJ // NW
IDX_PER_W = OBJ_PER_W * K
CHUNK = 128
NCHUNK = IDX_PER_W // CHUNK

_CONTRACT0 = (((0,), (0,)), ((), ()))


def _entity_tile(valsT_ref, rtq_ref, wf_ref, bf_ref, wa_ref, ba_ref,
                 went_ref, atnT_ref, outT_ref, el2_ref):
    i = pl.program_id(0)

    vt = valsT_ref[...]
    vtb = vt.astype(jnp.bfloat16)
    v_rep = jnp.broadcast_to(vtb[:, None, :], (A, H, BN)).reshape(AH, BN)
    th = jnp.tanh(v_rep * wf_ref[...] + bf_ref[...])
    scoresT = lax.dot_general(rtq_ref[...], th, _CONTRACT0,
                              preferred_element_type=jnp.float32)
    m = jnp.max(scoresT, axis=0, keepdims=True)
    e = jnp.exp(scoresT - m)
    wT = e / jnp.sum(e, axis=0, keepdims=True)
    elT = (lax.dot_general(wa_ref[...], wT * vt, _CONTRACT0,
                           preferred_element_type=jnp.float32)
           + lax.dot_general(ba_ref[...], wT, _CONTRACT0,
                             preferred_element_type=jnp.float32))
    outT_ref[...] = elT
    el2 = lax.dot_general(elT, went_ref[...], _CONTRACT0,
                          preferred_element_type=jnp.float32)
    el2_ref[...] = jnp.concatenate([el2, jnp.zeros_like(el2)], axis=1)

    @pl.when(i == GN)
    def _():
        outT_ref[:, 0:16] = atnT_ref[...]


def _entity_lookup(attr_vals, W_attr, b_attr, q_attn, W_ent, action_table):
    valsT = attr_vals.T
    wf = W_attr.reshape(AH, 1).astype(jnp.bfloat16)
    bf = b_attr.reshape(AH, 1).astype(jnp.bfloat16)
    rtq = (jnp.eye(A, dtype=jnp.float32)[:, None, :]
           * q_attn.reshape(1, H, 1)).reshape(AH, A).astype(jnp.bfloat16)
    atnT = jnp.concatenate(
        [action_table.T, jnp.zeros((H, 16 - NACT), jnp.float32)], axis=1)
    clamp = lambda i: (0, jnp.minimum(i, GN - 1))
    full = lambda *shape: pl.BlockSpec(shape, lambda i: (0,) * len(shape))
    return pl.pallas_call(
        _entity_tile,
        grid=(GN + 1,),
        in_specs=[
            pl.BlockSpec((A, BN), clamp),
            full(AH, A), full(AH, 1), full(AH, 1),
            full(A, H), full(A, H), full(H, H), full(H, 16),
        ],
        out_specs=[
            pl.BlockSpec((H, BN), lambda i: (0, i)),
            pl.BlockSpec((BN, 2 * H), lambda i: (jnp.minimum(i, GN - 1), 0)),
        ],
        out_shape=[
            jax.ShapeDtypeStruct((H, NFULL), jnp.float32),
            jax.ShapeDtypeStruct((N, 2 * H), jnp.float32),
        ],
    )(valsT, rtq, wf, bf, W_attr, b_attr, W_ent, atnT)


def _gather_body(el2_hbm, idx_hbm, out_hbm, idx_v, rows_v, pooled_v, sem):
    wid = lax.axis_index("s") * 2 + lax.axis_index("c")
    pltpu.sync_copy(idx_hbm.at[wid], idx_v)
    copies = [
        pltpu.async_copy(el2_hbm.at[idx_v.at[j]],
                         rows_v.at[pl.ds(j * CHUNK, CHUNK)], sem)
        for j in range(NCHUNK)
    ]
    for c in copies:
        c.wait()

    inv_k = jnp.float32(1.0 / K)

    def body(o, carry):
        for c in range(H // 16):
            acc = rows_v[o * K, pl.ds(c * 16, 16)]
            for k in range(1, K):
                acc = acc + rows_v[o * K + k, pl.ds(c * 16, 16)]
            y = acc * inv_k
            t = 1.0 - 2.0 / (jnp.exp(2.0 * y) + 1.0)
            pooled_v[o, pl.ds(c * 16, 16)] = t
        return carry

    lax.fori_loop(0, OBJ_PER_W, body, 0, unroll=False)
    pltpu.sync_copy(pooled_v, out_hbm.at[pl.ds(wid * OBJ_PER_W, OBJ_PER_W)])


def _gather_pool(el2, idx3d):
    mesh = plsc.VectorSubcoreMesh(core_axis_name="c", subcore_axis_name="s")
    fn = functools.partial(
        pl.kernel,
        mesh=mesh,
        out_type=jax.ShapeDtypeStruct((NOBJ, H), jnp.float32),
        scratch_types=[
            pltpu.VMEM((NCHUNK, CHUNK), jnp.int32),
            pltpu.VMEM((IDX_PER_W, 2 * H), jnp.float32),
            pltpu.VMEM((OBJ_PER_W, H), jnp.float32),
            pltpu.SemaphoreType.DMA,
        ],
    )(_gather_body)
    return fn(el2, idx3d)


def kernel(attr_vals, obs_idx, W_attr, b_attr, q_attn, W_ent, action_table):
    el_fullT, el2 = _entity_lookup(attr_vals, W_attr, b_attr, q_attn,
                                   W_ent, action_table)
    idx3d = obs_idx.astype(jnp.int32).reshape(NW, NCHUNK, CHUNK)
    obs = _gather_pool(el2, idx3d)
    return obs, el_fullT.T
